# baseline (speedup 1.0000x reference)
import jax
import jax.numpy as jnp
import numpy as np
from jax import lax
from jax.experimental import pallas as pl
from jax.experimental.pallas import tpu as pltpu
from jax.experimental.pallas import tpu_sc as plsc

N_NODES = 10000
N_EDGES = 320000
D = 128

NC = 2
NS = 16
NW = NC * NS
CH = 80
NCHUNK = 128
NHALF = 4
HALF = NCHUNK // NHALF
NBUF = 4
EPW = NCHUNK * CH
E_PAD = NW * EPW
N_ACC = 10112
ROWS_SUB = N_ACC // NS


def _sc_body(feat_hbm, src_hbm, dst_hbm, zero_hbm, out_hbm,
             src_v, dst_v, rows_v, acc, sem0, sem1, sem2, sem3, init_sem):
    sems = (sem0, sem1, sem2, sem3)
    c = lax.axis_index("c")
    s = lax.axis_index("s")
    w = c * NS + s

    init_cp = pltpu.async_copy(
        zero_hbm.at[pl.ds(s * ROWS_SUB, ROWS_SUB)],
        acc.at[pl.ds(s * ROWS_SUB, ROWS_SUB)], init_sem)
    pltpu.sync_copy(src_hbm.at[w].at[0], src_v)
    pltpu.sync_copy(dst_hbm.at[w].at[0], dst_v)
    for b in range(NBUF):
        pltpu.async_copy(feat_hbm.at[src_v.at[b]], rows_v.at[b], sems[b])
    init_cp.wait()
    plsc.subcore_barrier()

    for ph in range(NHALF):
        if ph > 0:
            pltpu.sync_copy(src_hbm.at[w].at[ph], src_v)
            pltpu.sync_copy(dst_hbm.at[w].at[ph], dst_v)
            for b in range(NBUF):
                pltpu.async_copy(
                    feat_hbm.at[src_v.at[b]], rows_v.at[b], sems[b])

        @pl.loop(0, (HALF - NBUF) // NBUF)
        def _grp(g):
            base = g * NBUF
            for b in range(NBUF):
                j = base + b
                pltpu.make_async_copy(
                    feat_hbm.at[src_v.at[j]], rows_v.at[b], sems[b]).wait()
                pltpu.sync_copy(rows_v.at[b], acc.at[dst_v.at[j]], add=True)
                pltpu.async_copy(
                    feat_hbm.at[src_v.at[j + NBUF]], rows_v.at[b], sems[b])

        for b in range(NBUF):
            j = HALF - NBUF + b
            pltpu.make_async_copy(
                feat_hbm.at[src_v.at[j]], rows_v.at[b], sems[b]).wait()
            pltpu.sync_copy(rows_v.at[b], acc.at[dst_v.at[j]], add=True)

    plsc.subcore_barrier()

    pltpu.sync_copy(acc.at[pl.ds(s * ROWS_SUB, ROWS_SUB)],
                    out_hbm.at[c].at[pl.ds(s * ROWS_SUB, ROWS_SUB)])


_sc_scatter = pl.kernel(
    _sc_body,
    out_type=jax.ShapeDtypeStruct((NC, N_ACC, D), jnp.float32),
    mesh=plsc.VectorSubcoreMesh(core_axis_name="c", subcore_axis_name="s"),
    scratch_types=[
        pltpu.VMEM((HALF, CH), jnp.int32),
        pltpu.VMEM((HALF, CH), jnp.int32),
        pltpu.VMEM((NBUF, CH, D), jnp.float32),
        pltpu.VMEM_SHARED((N_ACC, D), jnp.float32),
        pltpu.SemaphoreType.DMA,
        pltpu.SemaphoreType.DMA,
        pltpu.SemaphoreType.DMA,
        pltpu.SemaphoreType.DMA,
        pltpu.SemaphoreType.DMA,
    ],
)


def _tc_body(p_ref, w_ref, b_ref, o_ref):
    h = p_ref[0] + p_ref[1]
    o_ref[...] = lax.dot_general(
        h, w_ref[...], (((1,), (1,)), ((), ())),
        preferred_element_type=jnp.float32) + b_ref[...]


_ROWS_BLK = 1000
_tc_proj = pl.pallas_call(
    _tc_body,
    grid=(N_NODES // _ROWS_BLK,),
    in_specs=[
        pl.BlockSpec((NC, _ROWS_BLK, D), lambda i: (0, i, 0)),
        pl.BlockSpec((D, D), lambda i: (0, 0)),
        pl.BlockSpec((1, D), lambda i: (0, 0)),
    ],
    out_specs=pl.BlockSpec((_ROWS_BLK, D), lambda i: (i, 0)),
    out_shape=jax.ShapeDtypeStruct((N_NODES, D), jnp.float32),
)


_EPW_REAL = N_EDGES // NW
_PAD = EPW - _EPW_REAL
_PAD_SRC = (np.arange(NW * _PAD, dtype=np.int32) % N_NODES).reshape(NW, _PAD)
_PAD_DST = np.broadcast_to(
    N_NODES + (np.arange(_PAD, dtype=np.int32) % (N_ACC - N_NODES)),
    (NW, _PAD)).copy()
_ZEROS = np.zeros((N_ACC, D), np.float32)


def kernel(feature, edge_index, W, b):
    src = edge_index[0].astype(jnp.int32)
    dst = edge_index[1].astype(jnp.int32)
    src_p = jnp.concatenate(
        [src.reshape(NW, _EPW_REAL), _PAD_SRC],
        axis=1).reshape(NW, NHALF, HALF, CH)
    dst_p = jnp.concatenate(
        [dst.reshape(NW, _EPW_REAL), _PAD_DST],
        axis=1).reshape(NW, NHALF, HALF, CH)
    partials = _sc_scatter(feature, src_p, dst_p, _ZEROS)
    return _tc_proj(partials, W, b.reshape(1, D))


# speedup vs baseline: 1.0031x; 1.0031x over previous
import jax
import jax.numpy as jnp
import numpy as np
from jax import lax
from jax.experimental import pallas as pl
from jax.experimental.pallas import tpu as pltpu
from jax.experimental.pallas import tpu_sc as plsc

N_NODES = 10000
N_EDGES = 320000
D = 128

NC = 2
NS = 16
NW = NC * NS
CH = 80
NCHUNK = 128
NHALF = 4
HALF = NCHUNK // NHALF
NBUF = 4
EPW = NCHUNK * CH
E_PAD = NW * EPW
N_ACC = 10112
ROWS_SUB = N_ACC // NS


def _sc_body(feat_hbm, src_hbm, dst_hbm, src3_hbm, dst3_hbm, zero_hbm,
             out_hbm,
             src_v, dst_v, rows_v, acc, sem0, sem1, sem2, sem3, init_sem):
    sems = (sem0, sem1, sem2, sem3)
    c = lax.axis_index("c")
    s = lax.axis_index("s")
    w = c * NS + s

    init_cp = pltpu.async_copy(
        zero_hbm.at[pl.ds(s * ROWS_SUB, ROWS_SUB)],
        acc.at[pl.ds(s * ROWS_SUB, ROWS_SUB)], init_sem)
    pltpu.sync_copy(src_hbm.at[w].at[pl.ds(0, HALF)], src_v)
    pltpu.sync_copy(dst_hbm.at[w].at[pl.ds(0, HALF)], dst_v)
    for b in range(NBUF):
        pltpu.async_copy(feat_hbm.at[src_v.at[b]], rows_v.at[b], sems[b])
    init_cp.wait()
    plsc.subcore_barrier()

    for ph in range(NHALF):
        if ph > 0:
            if ph < NHALF - 1:
                pltpu.sync_copy(src_hbm.at[w].at[pl.ds(ph * HALF, HALF)],
                                src_v)
                pltpu.sync_copy(dst_hbm.at[w].at[pl.ds(ph * HALF, HALF)],
                                dst_v)
            else:
                pltpu.sync_copy(src3_hbm.at[w], src_v)
                pltpu.sync_copy(dst3_hbm.at[w], dst_v)
            for b in range(NBUF):
                pltpu.async_copy(
                    feat_hbm.at[src_v.at[b]], rows_v.at[b], sems[b])

        @pl.loop(0, (HALF - NBUF) // NBUF)
        def _grp(g):
            base = g * NBUF
            for b in range(NBUF):
                j = base + b
                pltpu.make_async_copy(
                    feat_hbm.at[src_v.at[j]], rows_v.at[b], sems[b]).wait()
                pltpu.sync_copy(rows_v.at[b], acc.at[dst_v.at[j]], add=True)
                pltpu.async_copy(
                    feat_hbm.at[src_v.at[j + NBUF]], rows_v.at[b], sems[b])

        for b in range(NBUF):
            j = HALF - NBUF + b
            pltpu.make_async_copy(
                feat_hbm.at[src_v.at[j]], rows_v.at[b], sems[b]).wait()
            pltpu.sync_copy(rows_v.at[b], acc.at[dst_v.at[j]], add=True)

    plsc.subcore_barrier()

    pltpu.sync_copy(acc.at[pl.ds(s * ROWS_SUB, ROWS_SUB)],
                    out_hbm.at[c].at[pl.ds(s * ROWS_SUB, ROWS_SUB)])


_sc_scatter = pl.kernel(
    _sc_body,
    out_type=jax.ShapeDtypeStruct((NC, N_ACC, D), jnp.float32),
    mesh=plsc.VectorSubcoreMesh(core_axis_name="c", subcore_axis_name="s"),
    scratch_types=[
        pltpu.VMEM((HALF, CH), jnp.int32),
        pltpu.VMEM((HALF, CH), jnp.int32),
        pltpu.VMEM((NBUF, CH, D), jnp.float32),
        pltpu.VMEM_SHARED((N_ACC, D), jnp.float32),
        pltpu.SemaphoreType.DMA,
        pltpu.SemaphoreType.DMA,
        pltpu.SemaphoreType.DMA,
        pltpu.SemaphoreType.DMA,
        pltpu.SemaphoreType.DMA,
    ],
)


def _tc_body(p_ref, w_ref, b_ref, o_ref):
    h = p_ref[0] + p_ref[1]
    o_ref[...] = lax.dot_general(
        h, w_ref[...], (((1,), (1,)), ((), ())),
        preferred_element_type=jnp.float32) + b_ref[...]


_ROWS_BLK = 1000
_tc_proj = pl.pallas_call(
    _tc_body,
    grid=(N_NODES // _ROWS_BLK,),
    in_specs=[
        pl.BlockSpec((NC, _ROWS_BLK, D), lambda i: (0, i, 0)),
        pl.BlockSpec((D, D), lambda i: (0, 0)),
        pl.BlockSpec((1, D), lambda i: (0, 0)),
    ],
    out_specs=pl.BlockSpec((_ROWS_BLK, D), lambda i: (i, 0)),
    out_shape=jax.ShapeDtypeStruct((N_NODES, D), jnp.float32),
)


_EPW_REAL = N_EDGES // NW
_NCH_REAL = _EPW_REAL // CH
_NCH_MAIN = (NHALF - 1) * HALF
_PAD = EPW - _EPW_REAL
_PAD_SRC = (np.arange(NW * _PAD, dtype=np.int32) % N_NODES).reshape(NW, _PAD)
_PAD_DST = np.broadcast_to(
    N_NODES + (np.arange(_PAD, dtype=np.int32) % (N_ACC - N_NODES)),
    (NW, _PAD)).copy()
_ZEROS = np.zeros((N_ACC, D), np.float32)


def kernel(feature, edge_index, W, b):
    src = edge_index[0].astype(jnp.int32).reshape(NW, _NCH_REAL, CH)
    dst = edge_index[1].astype(jnp.int32).reshape(NW, _NCH_REAL, CH)
    src3 = jnp.concatenate(
        [src[:, _NCH_MAIN:].reshape(NW, -1), _PAD_SRC],
        axis=1).reshape(NW, HALF, CH)
    dst3 = jnp.concatenate(
        [dst[:, _NCH_MAIN:].reshape(NW, -1), _PAD_DST],
        axis=1).reshape(NW, HALF, CH)
    partials = _sc_scatter(feature, src, dst, src3, dst3, _ZEROS)
    return _tc_proj(partials, W, b.reshape(1, D))


# speedup vs baseline: 1.0238x; 1.0206x over previous
import jax
import jax.numpy as jnp
import numpy as np
from jax import lax
from jax.experimental import pallas as pl
from jax.experimental.pallas import tpu as pltpu
from jax.experimental.pallas import tpu_sc as plsc

N_NODES = 10000
N_EDGES = 320000
D = 128

NC = 2
NS = 16
NW = NC * NS
CH = 80
NCHUNK = 128
NHALF = 4
HALF = NCHUNK // NHALF
NBUF = 4
EPW = NCHUNK * CH
E_PAD = NW * EPW
N_ACC = 10112
ROWS_SUB = N_ACC // NS


def _sc_body(feat_hbm, src_hbm, dst_hbm, src3_hbm, dst3_hbm, zero_hbm,
             out_hbm,
             src_v, dst_v, rows_v, acc, sem0, sem1, sem2, sem3, init_sem):
    sems = (sem0, sem1, sem2, sem3)
    c = lax.axis_index("c")
    s = lax.axis_index("s")
    w = c * NS + s

    init_cp = pltpu.async_copy(
        zero_hbm.at[pl.ds(s * ROWS_SUB, ROWS_SUB)],
        acc.at[pl.ds(s * ROWS_SUB, ROWS_SUB)], init_sem)
    pltpu.sync_copy(src_hbm.at[w].at[pl.ds(0, HALF)], src_v)
    pltpu.sync_copy(dst_hbm.at[w].at[pl.ds(0, HALF)], dst_v)
    for b in range(NBUF):
        pltpu.async_copy(feat_hbm.at[src_v.at[b]], rows_v.at[b], sems[b])
    init_cp.wait()
    plsc.subcore_barrier()

    for ph in range(NHALF):
        if ph > 0:
            if ph < NHALF - 1:
                pltpu.sync_copy(src_hbm.at[w].at[pl.ds(ph * HALF, HALF)],
                                src_v)
                pltpu.sync_copy(dst_hbm.at[w].at[pl.ds(ph * HALF, HALF)],
                                dst_v)
            else:
                pltpu.sync_copy(src3_hbm.at[w], src_v)
                pltpu.sync_copy(dst3_hbm.at[w], dst_v)
            for b in range(NBUF):
                pltpu.async_copy(
                    feat_hbm.at[src_v.at[b]], rows_v.at[b], sems[b])

        @pl.loop(0, (HALF - NBUF) // NBUF)
        def _grp(g):
            base = g * NBUF
            for b in range(NBUF):
                j = base + b
                pltpu.make_async_copy(
                    feat_hbm.at[src_v.at[j]], rows_v.at[b], sems[b]).wait()
                pltpu.sync_copy(rows_v.at[b], acc.at[dst_v.at[j]], add=True)
                pltpu.async_copy(
                    feat_hbm.at[src_v.at[j + NBUF]], rows_v.at[b], sems[b])

        for b in range(NBUF):
            j = HALF - NBUF + b
            pltpu.make_async_copy(
                feat_hbm.at[src_v.at[j]], rows_v.at[b], sems[b]).wait()
            pltpu.sync_copy(rows_v.at[b], acc.at[dst_v.at[j]], add=True)

    plsc.subcore_barrier()

    pltpu.sync_copy(acc.at[pl.ds(s * ROWS_SUB, ROWS_SUB)],
                    out_hbm.at[c].at[pl.ds(s * ROWS_SUB, ROWS_SUB)])


_sc_scatter = pl.kernel(
    _sc_body,
    out_type=jax.ShapeDtypeStruct((NC, N_ACC, D), jnp.float32),
    mesh=plsc.VectorSubcoreMesh(core_axis_name="c", subcore_axis_name="s"),
    scratch_types=[
        pltpu.VMEM((HALF, CH), jnp.int32),
        pltpu.VMEM((HALF, CH), jnp.int32),
        pltpu.VMEM((NBUF, CH, D), jnp.float32),
        pltpu.VMEM_SHARED((N_ACC, D), jnp.float32),
        pltpu.SemaphoreType.DMA,
        pltpu.SemaphoreType.DMA,
        pltpu.SemaphoreType.DMA,
        pltpu.SemaphoreType.DMA,
        pltpu.SemaphoreType.DMA,
    ],
)


def _tc_body(p_ref, w_ref, b_ref, o_ref):
    h = p_ref[0] + p_ref[1]
    o_ref[...] = lax.dot_general(
        h, w_ref[...], (((1,), (1,)), ((), ())),
        preferred_element_type=jnp.float32) + b_ref[...]


_ROWS_BLK = 2000
_tc_proj = pl.pallas_call(
    _tc_body,
    grid=(N_NODES // _ROWS_BLK,),
    in_specs=[
        pl.BlockSpec((NC, _ROWS_BLK, D), lambda i: (0, i, 0)),
        pl.BlockSpec((D, D), lambda i: (0, 0)),
        pl.BlockSpec((1, D), lambda i: (0, 0)),
    ],
    out_specs=pl.BlockSpec((_ROWS_BLK, D), lambda i: (i, 0)),
    out_shape=jax.ShapeDtypeStruct((N_NODES, D), jnp.float32),
)


_EPW_REAL = N_EDGES // NW
_NCH_REAL = _EPW_REAL // CH
_NCH_MAIN = (NHALF - 1) * HALF
_PAD = EPW - _EPW_REAL
_PAD_SRC = (np.arange(NW * _PAD, dtype=np.int32) % N_NODES).reshape(NW, _PAD)
_PAD_DST = np.broadcast_to(
    N_NODES + (np.arange(_PAD, dtype=np.int32) % (N_ACC - N_NODES)),
    (NW, _PAD)).copy()
_ZEROS = np.zeros((N_ACC, D), np.float32)


def kernel(feature, edge_index, W, b):
    src = edge_index[0].astype(jnp.int32).reshape(NW, _NCH_REAL, CH)
    dst = edge_index[1].astype(jnp.int32).reshape(NW, _NCH_REAL, CH)
    src3 = jnp.concatenate(
        [src[:, _NCH_MAIN:].reshape(NW, -1), _PAD_SRC],
        axis=1).reshape(NW, HALF, CH)
    dst3 = jnp.concatenate(
        [dst[:, _NCH_MAIN:].reshape(NW, -1), _PAD_DST],
        axis=1).reshape(NW, HALF, CH)
    partials = _sc_scatter(feature, src, dst, src3, dst3, _ZEROS)
    return _tc_proj(partials, W, b.reshape(1, D))
